# Initial kernel scaffold; baseline (speedup 1.0000x reference)
import jax
import jax.numpy as jnp
from jax.experimental import pallas as pl


def kernel(x, edge_index, W, b):
    raise NotImplementedError("write your pallas kernel here")


# speedup vs baseline: 9.1498x; 9.1498x over previous
import functools

import jax
import jax.numpy as jnp
from jax import lax
from jax.experimental import pallas as pl
from jax.experimental.pallas import tpu as pltpu
from jax.experimental.pallas import tpu_sc as plsc


def _transform_body(x_ref, w_ref, b_ref, o_ref):
    x = x_ref[...]
    nrm = jnp.sqrt(jnp.sum(x * x, axis=1, keepdims=True))
    y = jnp.minimum(nrm, 1.0 - 1e-7)
    artanh = 0.5 * jnp.log((1.0 + y) / (1.0 - y))
    t = x * (artanh / jnp.maximum(nrm, 1e-15))
    o_ref[...] = (
        lax.dot_general(t, w_ref[...], (((1,), (1,)), ((), ())),
                        preferred_element_type=jnp.float32)
        + b_ref[...]
    )


def _transform(x, w, b, block_n):
    n, d_in = x.shape
    d_out = w.shape[0]
    return pl.pallas_call(
        _transform_body,
        grid=(n // block_n,),
        in_specs=[
            pl.BlockSpec((block_n, d_in), lambda i: (i, 0)),
            pl.BlockSpec((d_out, d_in), lambda i: (0, 0)),
            pl.BlockSpec((1, d_out), lambda i: (0, 0)),
        ],
        out_specs=pl.BlockSpec((block_n, d_out), lambda i: (i, 0)),
        out_shape=jax.ShapeDtypeStruct((n, d_out), jnp.float32),
    )(x, w, b.reshape(1, d_out))


def _make_seg_sum(n, d, nw, n_chunks, chunk):
    mesh = plsc.VectorSubcoreMesh(core_axis_name="c", subcore_axis_name="s")
    ns = 16
    npt = (n // ns) // 8 * 8
    rem = n - ns * npt

    @functools.partial(
        pl.kernel,
        mesh=mesh,
        out_type=(
            jax.ShapeDtypeStruct((2, n, d), jnp.float32),
            jax.ShapeDtypeStruct((2, n), jnp.float32),
        ),
        scratch_types=[
            pltpu.VMEM_SHARED((n, d), jnp.float32),
            pltpu.VMEM_SHARED((n,), jnp.float32),
            pltpu.VMEM((n_chunks, chunk), jnp.int32),
            pltpu.VMEM((n_chunks, chunk), jnp.int32),
            pltpu.VMEM((chunk, d), jnp.float32),
            pltpu.VMEM((128,), jnp.float32),
            pltpu.SemaphoreType.DMA,
        ],
    )
    def seg_sum(t_hbm, src_hbm, dst_hbm, z2d_hbm, z1d_hbm,
                agg_hbm, cnt_hbm,
                acc_sh, cnt_sh, sidx_v, didx_v, rows_v, ones_v, sem):
        cc = lax.axis_index("c")
        s = lax.axis_index("s")
        wid = cc * ns + s

        pltpu.sync_copy(src_hbm.at[wid], sidx_v)
        pltpu.sync_copy(dst_hbm.at[wid], didx_v)

        pltpu.sync_copy(z2d_hbm.at[pl.ds(s * npt, npt)],
                        acc_sh.at[pl.ds(s * npt, npt)])

        @pl.when(s == 0)
        def _():
            pltpu.sync_copy(z1d_hbm, cnt_sh)
            if rem:
                pltpu.sync_copy(z2d_hbm.at[pl.ds(ns * npt, rem)],
                                acc_sh.at[pl.ds(ns * npt, rem)])

        for i in range(8):
            ones_v[pl.ds(i * 16, 16)] = jnp.full((16,), 1.0, jnp.float32)

        plsc.subcore_barrier()

        def body(j, carry):
            pltpu.async_copy(t_hbm.at[sidx_v.at[j]], rows_v, sem).wait()
            pltpu.sync_copy(rows_v, acc_sh.at[didx_v.at[j]], add=True)
            pltpu.sync_copy(ones_v.at[pl.ds(0, chunk)],
                            cnt_sh.at[didx_v.at[j]], add=True)
            return carry

        lax.fori_loop(0, n_chunks, body, 0)

        plsc.subcore_barrier()

        pltpu.sync_copy(acc_sh.at[pl.ds(s * npt, npt)],
                        agg_hbm.at[cc, pl.ds(s * npt, npt)])

        @pl.when(s == 0)
        def _():
            pltpu.sync_copy(cnt_sh, cnt_hbm.at[cc])
            if rem:
                pltpu.sync_copy(acc_sh.at[pl.ds(ns * npt, rem)],
                                agg_hbm.at[cc, pl.ds(ns * npt, rem)])

    return seg_sum


def _finalize_body(agg_ref, cnt_ref, o_ref):
    a = agg_ref[0] + agg_ref[1]
    c = cnt_ref[...]
    csum = c[:, 0:1] + c[:, 1:2]
    neigh = a / jnp.maximum(csum, 1.0)
    nrm = jnp.sqrt(jnp.sum(neigh * neigh, axis=1, keepdims=True))
    o_ref[...] = jnp.tanh(nrm) * neigh / jnp.maximum(nrm, 1e-15)


def _finalize(agg, cnt_t, block_n):
    _, n, d = agg.shape
    return pl.pallas_call(
        _finalize_body,
        grid=(n // block_n,),
        in_specs=[
            pl.BlockSpec((2, block_n, d), lambda i: (0, i, 0)),
            pl.BlockSpec((block_n, 2), lambda i: (i, 0)),
        ],
        out_specs=pl.BlockSpec((block_n, d), lambda i: (i, 0)),
        out_shape=jax.ShapeDtypeStruct((n, d), jnp.float32),
    )(agg, cnt_t)


def kernel(x, edge_index, W, b):
    n, d_in = x.shape
    d_out = W.shape[0]
    e = edge_index.shape[1]

    nw = 32
    chunk = 125
    epw = e // nw
    n_chunks = epw // chunk
    assert epw * nw == e and n_chunks * chunk == epw

    transformed = _transform(x, W, b, block_n=1000)

    src = edge_index[0].reshape(nw, n_chunks, chunk)
    dst = edge_index[1].reshape(nw, n_chunks, chunk)
    z2d = jnp.zeros((n, d_out), jnp.float32)
    z1d = jnp.zeros((n,), jnp.float32)

    seg_sum = _make_seg_sum(n, d_out, nw, n_chunks, chunk)
    agg, cnt = seg_sum(transformed, src, dst, z2d, z1d)

    return _finalize(agg, cnt.T, block_n=1000)
